# baseline (speedup 1.0000x reference)
import functools

import jax
import jax.numpy as jnp
from jax import lax
from jax.experimental import pallas as pl
from jax.experimental.pallas import tpu as pltpu
from jax.experimental.pallas import tpu_sc as plsc

N_NODES = 10000
N_EDGES = 320000
D_FEAT = 128
NC = 2
NS = 16
NW = NC * NS
EPT = N_EDGES // NW
KC = 40
NCH = EPT // KC
NBUF = 3
RPT = 632
N_PAD = NS * RPT


def _agg_body(h_hbm, send_hbm, recv_hbm, znd_hbm, zn_hbm, ones_hbm,
              a_out, deg_out,
              sidx_all, ridx0, ridx1, ridx2,
              rows0, rows1, rows2,
              ones_v, zbuf, acc, dacc,
              sg0, sg1, sg2, ss0, ss1, ss2,
              sd0, sd1, sd2, sr0, sr1, sr2):
    cid = lax.axis_index("c")
    sid = lax.axis_index("s")
    wid = sid * NC + cid
    ebase = wid * EPT
    rbase = sid * RPT

    ridx = (ridx0, ridx1, ridx2)
    rows = (rows0, rows1, rows2)
    sg = (sg0, sg1, sg2)
    ss = (ss0, ss1, ss2)
    sd = (sd0, sd1, sd2)
    sr = (sr0, sr1, sr2)

    def fire_gather(g, b):
        pltpu.async_copy(recv_hbm.at[pl.ds(ebase + g * KC, KC)],
                         ridx[b], sr[b])
        pltpu.async_copy(h_hbm.at[sidx_all.at[pl.ds(g * KC, KC)]],
                         rows[b], sg[b])

    def wait_gather(g, b):
        pltpu.make_async_copy(recv_hbm.at[pl.ds(ebase + g * KC, KC)],
                              ridx[b], sr[b]).wait()
        pltpu.make_async_copy(h_hbm.at[sidx_all.at[pl.ds(g * KC, KC)]],
                              rows[b], sg[b]).wait()

    def fire_scatter(g, b):
        pltpu.async_copy(rows[b], acc.at[ridx[b]], ss[b], add=True)
        pltpu.async_copy(ones_v, dacc.at[ridx[b]], sd[b], add=True)

    def wait_scatter(g, b):
        pltpu.make_async_copy(rows[b], acc.at[ridx[b]], ss[b]).wait()
        pltpu.make_async_copy(ones_v, dacc.at[ridx[b]], sd[b]).wait()

    pltpu.sync_copy(znd_hbm.at[pl.ds(rbase, RPT)], acc.at[pl.ds(rbase, RPT)])
    pltpu.sync_copy(zn_hbm.at[pl.ds(rbase, RPT)], zbuf)
    pltpu.sync_copy(zbuf, dacc.at[pl.ds(rbase, RPT)])
    pltpu.sync_copy(send_hbm.at[pl.ds(ebase, EPT)], sidx_all)
    pltpu.sync_copy(ones_hbm, ones_v)

    for g in range(NBUF - 1):
        fire_gather(g, g)
    plsc.subcore_barrier()

    def visit(i, b):
        v = NBUF * i + b
        s_next = (b + NBUF - 1) % NBUF

        @pl.when(v + NBUF - 1 < NCH)
        def _():
            if b == 0:
                @pl.when(i > 0)
                def _():
                    wait_scatter(v - 1, s_next)
            else:
                wait_scatter(v - 1, s_next)
            fire_gather(v + NBUF - 1, s_next)

        wait_gather(v, b)
        fire_scatter(v, b)

    def group(i, carry):
        for b in range(NBUF):
            visit(i, b)
        return carry

    nfull = NCH // NBUF
    lax.fori_loop(0, nfull, group, 0)
    for g in range(nfull * NBUF, NCH):
        wait_gather(g, g % NBUF)
        fire_scatter(g, g % NBUF)
    for g in range(NCH - NBUF, NCH):
        wait_scatter(g, g % NBUF)

    plsc.subcore_barrier()

    pltpu.sync_copy(acc.at[pl.ds(rbase, RPT)],
                    a_out.at[pl.ds(cid * N_PAD + rbase, RPT)])
    pltpu.sync_copy(dacc.at[pl.ds(rbase, RPT)], zbuf)
    pltpu.sync_copy(zbuf, deg_out.at[pl.ds(cid * N_PAD + rbase, RPT)])


_agg_call = pl.kernel(
    _agg_body,
    out_type=[
        jax.ShapeDtypeStruct((NC * N_PAD, D_FEAT), jnp.float32),
        jax.ShapeDtypeStruct((NC * N_PAD,), jnp.float32),
    ],
    mesh=plsc.VectorSubcoreMesh(core_axis_name="c", subcore_axis_name="s"),
    scratch_types=(
        [pltpu.VMEM((EPT,), jnp.int32)]
        + [pltpu.VMEM((KC,), jnp.int32) for _ in range(NBUF)]
        + [pltpu.VMEM((KC, D_FEAT), jnp.float32) for _ in range(NBUF)]
        + [pltpu.VMEM((KC,), jnp.float32),
           pltpu.VMEM((RPT,), jnp.float32),
           pltpu.VMEM_SHARED((N_PAD, D_FEAT), jnp.float32),
           pltpu.VMEM_SHARED((N_PAD,), jnp.float32)]
        + [pltpu.SemaphoreType.DMA for _ in range(4 * NBUF)]
    ),
)


def _layer_body(h_ref, a_ref, degp_ref, ws_ref, wn_ref, b_ref, o_ref):
    degp = degp_ref[...]
    deg = degp[:N_NODES] + degp[N_PAD:N_PAD + N_NODES]
    inv = 1.0 / jnp.maximum(deg, 1.0)
    agg = (a_ref[:N_NODES, :] + a_ref[N_PAD:N_PAD + N_NODES, :]) * inv[:, None]
    o_ref[...] = jax.nn.relu(
        jnp.dot(h_ref[...], ws_ref[...], preferred_element_type=jnp.float32)
        + jnp.dot(agg, wn_ref[...], preferred_element_type=jnp.float32)
        + b_ref[...][None, :])


_layer_call = pl.pallas_call(
    _layer_body,
    out_shape=jax.ShapeDtypeStruct((N_NODES, D_FEAT), jnp.float32),
)


def _qeq_body(h_ref, a_ref, degp_ref, ws_ref, wn_ref, b_ref,
              wr_ref, br_ref, sid_ref, tq_ref, q_ref):
    n = N_NODES
    g = tq_ref.shape[0]
    degp = degp_ref[...]
    deg = degp[:N_NODES] + degp[N_PAD:N_PAD + N_NODES]
    inv_d = 1.0 / jnp.maximum(deg, 1.0)
    agg = (a_ref[:N_NODES, :] + a_ref[N_PAD:N_PAD + N_NODES, :]) * inv_d[:, None]
    h3 = jax.nn.relu(
        jnp.dot(h_ref[...], ws_ref[...], preferred_element_type=jnp.float32)
        + jnp.dot(agg, wn_ref[...], preferred_element_type=jnp.float32)
        + b_ref[...][None, :])
    es = lax.dot_general(h3, wr_ref[...],
                         (((1,), (1,)), ((), ())),
                         preferred_element_type=jnp.float32)
    es = es + br_ref[...][None, :]
    e = es[:, 0]
    s = es[:, 1]
    sp = jnp.maximum(s, 0.0) + jnp.log1p(jnp.exp(-jnp.abs(s))) + 1e-4
    inv = 0.5 / sp
    sid = sid_ref[...]
    gi = lax.broadcasted_iota(jnp.int32, (g, n), 0)
    oneh = (gi == sid[None, :]).astype(jnp.float32)
    vals = jnp.stack([inv, e * inv], axis=1)
    sums = jnp.dot(oneh, vals, preferred_element_type=jnp.float32)
    sum_inv = sums[:, 0]
    sum_e = sums[:, 1]
    lam = jnp.where(sum_inv > 0.0,
                    (tq_ref[...] + sum_e) / sum_inv, 0.0)
    lam_n = lax.dot_general(lam, oneh, (((0,), (0,)), ((), ())),
                            preferred_element_type=jnp.float32)
    q_ref[...] = (lam_n - e) * inv


def _qeq_call(h, a, degp, ws, wn, b, wr, br, sid, tq):
    return pl.pallas_call(
        _qeq_body,
        out_shape=jax.ShapeDtypeStruct((N_NODES,), jnp.float32),
    )(h, a, degp, ws, wn, b, wr, br, sid, tq)


def kernel(x, senders, receivers, segment_ids, num_graphs, total_charge,
           Ws1, Wn1, b1, Ws2, Wn2, b2, Ws3, Wn3, b3, w_readout, b_readout):
    znd = jnp.zeros((N_PAD, D_FEAT), jnp.float32)
    zn = jnp.zeros((N_PAD,), jnp.float32)
    ones_k = jnp.ones((KC,), jnp.float32)

    h = x
    for (Ws, Wn, b) in ((Ws1, Wn1, b1), (Ws2, Wn2, b2)):
        a, degp = _agg_call(h, senders, receivers, znd, zn, ones_k)
        h = _layer_call(h, a, degp, Ws, Wn, b)

    a, degp = _agg_call(h, senders, receivers, znd, zn, ones_k)
    q = _qeq_call(h, a, degp, Ws3, Wn3, b3,
                  w_readout, b_readout, segment_ids, total_charge)
    return q.reshape(N_NODES, 1)


# speedup vs baseline: 1.1954x; 1.1954x over previous
import functools

import jax
import jax.numpy as jnp
from jax import lax
from jax.experimental import pallas as pl
from jax.experimental.pallas import tpu as pltpu
from jax.experimental.pallas import tpu_sc as plsc

N_NODES = 10000
N_EDGES = 320000
D_FEAT = 128
NC = 2
NS = 16
NW = NC * NS
EPT = N_EDGES // NW
KC = 80
NCH = EPT // KC
NBUF = 3
RPT = 632
N_PAD = NS * RPT


def _make_agg_body(with_deg):
    def body(*refs):
        if with_deg:
            (h_hbm, send_hbm, recv_hbm, znd_hbm, zn_hbm, ones_hbm,
             a_out, deg_out,
             sidx_all, ridx0, ridx1, ridx2, rows0, rows1, rows2,
             ones_v, zbuf, acc, dacc, *sems) = refs
            sg, ss, sd, sr = (sems[0:3], sems[3:6], sems[6:9], sems[9:12])
        else:
            (h_hbm, send_hbm, recv_hbm, znd_hbm,
             a_out,
             sidx_all, ridx0, ridx1, ridx2, rows0, rows1, rows2,
             acc, *sems) = refs
            sg, ss, sr = (sems[0:3], sems[3:6], sems[6:9])

        cid = lax.axis_index("c")
        sid = lax.axis_index("s")
        wid = sid * NC + cid
        ebase = wid * EPT
        rbase = sid * RPT

        ridx = (ridx0, ridx1, ridx2)
        rows = (rows0, rows1, rows2)

        def fire_gather(g, b):
            pltpu.async_copy(recv_hbm.at[pl.ds(ebase + g * KC, KC)],
                             ridx[b], sr[b])
            pltpu.async_copy(h_hbm.at[sidx_all.at[pl.ds(g * KC, KC)]],
                             rows[b], sg[b])

        def wait_gather(g, b):
            pltpu.make_async_copy(recv_hbm.at[pl.ds(ebase + g * KC, KC)],
                                  ridx[b], sr[b]).wait()
            pltpu.make_async_copy(h_hbm.at[sidx_all.at[pl.ds(g * KC, KC)]],
                                  rows[b], sg[b]).wait()

        def fire_scatter(g, b):
            pltpu.async_copy(rows[b], acc.at[ridx[b]], ss[b], add=True)
            if with_deg:
                pltpu.async_copy(ones_v, dacc.at[ridx[b]], sd[b], add=True)

        def wait_scatter(g, b):
            pltpu.make_async_copy(rows[b], acc.at[ridx[b]], ss[b]).wait()
            if with_deg:
                pltpu.make_async_copy(ones_v, dacc.at[ridx[b]], sd[b]).wait()

        pltpu.sync_copy(znd_hbm.at[pl.ds(rbase, RPT)],
                        acc.at[pl.ds(rbase, RPT)])
        if with_deg:
            pltpu.sync_copy(zn_hbm.at[pl.ds(rbase, RPT)], zbuf)
            pltpu.sync_copy(zbuf, dacc.at[pl.ds(rbase, RPT)])
            pltpu.sync_copy(ones_hbm, ones_v)
        pltpu.sync_copy(send_hbm.at[pl.ds(ebase, EPT)], sidx_all)

        for g in range(NBUF - 1):
            fire_gather(g, g)
        plsc.subcore_barrier()

        def visit(i, b):
            v = NBUF * i + b
            s_next = (b + NBUF - 1) % NBUF

            @pl.when(v + NBUF - 1 < NCH)
            def _():
                if b == 0:
                    @pl.when(i > 0)
                    def _():
                        wait_scatter(v - 1, s_next)
                else:
                    wait_scatter(v - 1, s_next)
                fire_gather(v + NBUF - 1, s_next)

            wait_gather(v, b)
            fire_scatter(v, b)

        def group(i, carry):
            for b in range(NBUF):
                visit(i, b)
            return carry

        nfull = NCH // NBUF
        lax.fori_loop(0, nfull, group, 0)
        for g in range(nfull * NBUF, NCH):
            wait_gather(g, g % NBUF)
            fire_scatter(g, g % NBUF)
        for g in range(NCH - NBUF, NCH):
            wait_scatter(g, g % NBUF)

        plsc.subcore_barrier()

        pltpu.sync_copy(acc.at[pl.ds(rbase, RPT)],
                        a_out.at[pl.ds(cid * N_PAD + rbase, RPT)])
        if with_deg:
            pltpu.sync_copy(dacc.at[pl.ds(rbase, RPT)], zbuf)
            pltpu.sync_copy(zbuf, deg_out.at[pl.ds(cid * N_PAD + rbase, RPT)])

    return body


_A_TYPE = jax.ShapeDtypeStruct((NC * N_PAD, D_FEAT), jnp.float32)
_DEG_TYPE = jax.ShapeDtypeStruct((NC * N_PAD,), jnp.float32)
_MESH = plsc.VectorSubcoreMesh(core_axis_name="c", subcore_axis_name="s")
_COMMON_SCRATCH = (
    [pltpu.VMEM((EPT,), jnp.int32)]
    + [pltpu.VMEM((KC,), jnp.int32) for _ in range(NBUF)]
    + [pltpu.VMEM((KC, D_FEAT), jnp.float32) for _ in range(NBUF)]
)

_agg_deg_call = pl.kernel(
    _make_agg_body(True),
    out_type=[_A_TYPE, _DEG_TYPE],
    mesh=_MESH,
    scratch_types=(
        _COMMON_SCRATCH
        + [pltpu.VMEM((KC,), jnp.float32),
           pltpu.VMEM((RPT,), jnp.float32),
           pltpu.VMEM_SHARED((N_PAD, D_FEAT), jnp.float32),
           pltpu.VMEM_SHARED((N_PAD,), jnp.float32)]
        + [pltpu.SemaphoreType.DMA for _ in range(4 * NBUF)]
    ),
)

_agg_call = pl.kernel(
    _make_agg_body(False),
    out_type=_A_TYPE,
    mesh=_MESH,
    scratch_types=(
        _COMMON_SCRATCH
        + [pltpu.VMEM_SHARED((N_PAD, D_FEAT), jnp.float32)]
        + [pltpu.SemaphoreType.DMA for _ in range(3 * NBUF)]
    ),
)


def _layer_body(h_ref, a_ref, degp_ref, ws_ref, wn_ref, b_ref, o_ref):
    degp = degp_ref[...]
    deg = degp[:N_NODES] + degp[N_PAD:N_PAD + N_NODES]
    inv = 1.0 / jnp.maximum(deg, 1.0)
    agg = (a_ref[:N_NODES, :] + a_ref[N_PAD:N_PAD + N_NODES, :]) * inv[:, None]
    o_ref[...] = jax.nn.relu(
        jnp.dot(h_ref[...], ws_ref[...], preferred_element_type=jnp.float32)
        + jnp.dot(agg, wn_ref[...], preferred_element_type=jnp.float32)
        + b_ref[...][None, :])


_layer_call = pl.pallas_call(
    _layer_body,
    out_shape=jax.ShapeDtypeStruct((N_NODES, D_FEAT), jnp.float32),
)


def _qeq_body(h_ref, a_ref, degp_ref, ws_ref, wn_ref, b_ref,
              wr_ref, br_ref, sid_ref, tq_ref, q_ref):
    n = N_NODES
    g = tq_ref.shape[0]
    degp = degp_ref[...]
    deg = degp[:N_NODES] + degp[N_PAD:N_PAD + N_NODES]
    inv_d = 1.0 / jnp.maximum(deg, 1.0)
    agg = (a_ref[:N_NODES, :] + a_ref[N_PAD:N_PAD + N_NODES, :]) * inv_d[:, None]
    h3 = jax.nn.relu(
        jnp.dot(h_ref[...], ws_ref[...], preferred_element_type=jnp.float32)
        + jnp.dot(agg, wn_ref[...], preferred_element_type=jnp.float32)
        + b_ref[...][None, :])
    es = lax.dot_general(h3, wr_ref[...],
                         (((1,), (1,)), ((), ())),
                         preferred_element_type=jnp.float32)
    es = es + br_ref[...][None, :]
    e = es[:, 0]
    s = es[:, 1]
    sp = jnp.maximum(s, 0.0) + jnp.log1p(jnp.exp(-jnp.abs(s))) + 1e-4
    inv = 0.5 / sp
    sid = sid_ref[...]
    gi = lax.broadcasted_iota(jnp.int32, (g, n), 0)
    oneh = (gi == sid[None, :]).astype(jnp.float32)
    vals = jnp.stack([inv, e * inv], axis=1)
    sums = jnp.dot(oneh, vals, preferred_element_type=jnp.float32)
    sum_inv = sums[:, 0]
    sum_e = sums[:, 1]
    lam = jnp.where(sum_inv > 0.0,
                    (tq_ref[...] + sum_e) / sum_inv, 0.0)
    lam_n = lax.dot_general(lam, oneh, (((0,), (0,)), ((), ())),
                            preferred_element_type=jnp.float32)
    q_ref[...] = (lam_n - e) * inv


def _qeq_call(h, a, degp, ws, wn, b, wr, br, sid, tq):
    return pl.pallas_call(
        _qeq_body,
        out_shape=jax.ShapeDtypeStruct((N_NODES,), jnp.float32),
    )(h, a, degp, ws, wn, b, wr, br, sid, tq)


def kernel(x, senders, receivers, segment_ids, num_graphs, total_charge,
           Ws1, Wn1, b1, Ws2, Wn2, b2, Ws3, Wn3, b3, w_readout, b_readout):
    znd = jnp.zeros((N_PAD, D_FEAT), jnp.float32)
    zn = jnp.zeros((N_PAD,), jnp.float32)
    ones_k = jnp.ones((KC,), jnp.float32)

    a, degp = _agg_deg_call(x, senders, receivers, znd, zn, ones_k)
    h = _layer_call(x, a, degp, Ws1, Wn1, b1)

    a = _agg_call(h, senders, receivers, znd)
    h = _layer_call(h, a, degp, Ws2, Wn2, b2)

    a = _agg_call(h, senders, receivers, znd)
    q = _qeq_call(h, a, degp, Ws3, Wn3, b3,
                  w_readout, b_readout, segment_ids, total_charge)
    return q.reshape(N_NODES, 1)
